# Initial kernel scaffold; baseline (speedup 1.0000x reference)
import jax
import jax.numpy as jnp
from jax.experimental import pallas as pl


def kernel(z, W):
    raise NotImplementedError("write your pallas kernel here")


# speedup vs baseline: 1.4999x; 1.4999x over previous
import functools

import jax
import jax.numpy as jnp
from jax import lax
from jax.experimental import pallas as pl
from jax.experimental.pallas import tpu as pltpu
from jax.experimental.pallas import tpu_sc as plsc

N_EMB = 8192
EMB_DIM = 256
BATCH = 16
SEQ = 1024
M = BATCH * SEQ
BETA = 0.25

BL = 512
BK = 1024


_CHUNKS = ((0, 2736), (2736, 2736), (5472, 2720))


def _bf16_round(x):
    return x.astype(jnp.bfloat16).astype(jnp.float32)


def _dist_body(z_ref, w_ref, idx_ref, loss_ref, acc):
    zb = z_ref[0]
    z2 = jnp.sum(zb * zb, axis=0, keepdims=True)
    rmin = ridx = rsel = None
    for (s, c) in _CHUNKS:
        wb = w_ref[s:s + c]
        mm = lax.dot_general(wb, zb, (((1,), (0,)), ((), ())),
                             preferred_element_type=jnp.float32)
        w2 = jnp.sum(wb * wb, axis=1, keepdims=True)
        d = (z2 + w2) - 2.0 * mm
        lmin = jnp.min(d, axis=0, keepdims=True)
        iota = lax.broadcasted_iota(jnp.int32, (c, BL), 0)
        lidx = jnp.min(jnp.where(d == lmin, iota, N_EMB), axis=0,
                       keepdims=True) + s
        if rmin is None:
            ridx, rsel = lidx, lmin
            rmin = _bf16_round(lmin)
        else:
            keep = (rmin < lmin) | ((rmin == lmin) & (ridx < lidx))
            ridx = jnp.where(keep, ridx, lidx)
            rsel = jnp.where(keep, rsel, lmin)
            rmin = _bf16_round(jnp.where(keep, rmin, lmin))
    idx_ref[0] = ridx
    partial = jnp.sum(rsel)
    first = (pl.program_id(0) == 0) & (pl.program_id(1) == 0)
    total = jnp.where(first, partial, acc[0] + partial)
    acc[0] = total
    loss_ref[0, 0] = total * ((1.0 + BETA) / (M * EMB_DIM))


def _distance_argmin(z, w):
    grid = (BATCH, SEQ // BL)
    idx, loss = pl.pallas_call(
        _dist_body,
        grid=grid,
        in_specs=[
            pl.BlockSpec((1, EMB_DIM, BL), lambda b, l: (b, 0, l)),
            pl.BlockSpec((N_EMB, EMB_DIM), lambda b, l: (0, 0)),
        ],
        out_specs=[
            pl.BlockSpec((1, 1, BL), lambda b, l: (b * (SEQ // BL) + l, 0, 0)),
            pl.BlockSpec((1, 1), lambda b, l: (0, 0),
                         memory_space=pltpu.SMEM),
        ],
        out_shape=[
            jax.ShapeDtypeStruct((M // BL, 1, BL), jnp.int32),
            jax.ShapeDtypeStruct((1, 1), jnp.float32),
        ],
        scratch_shapes=[
            pltpu.SMEM((1,), jnp.float32),
        ],
    )(z, w)
    return idx.reshape(M), loss[0, 0]


_NC = 2
_NS = 16
_NW = _NC * _NS
_BPW = M // _NW
_CH = 128

@functools.lru_cache(maxsize=None)
def _build_sc_gather():
    mesh = plsc.VectorSubcoreMesh(core_axis_name="c", subcore_axis_name="s")

    @functools.partial(
        pl.kernel,
        mesh=mesh,
        out_type=jax.ShapeDtypeStruct((M, EMB_DIM), jnp.float32),
        scratch_types=[
            pltpu.VMEM((_CH,), jnp.int32),
            pltpu.VMEM((_CH, EMB_DIM), jnp.float32),
            pltpu.SemaphoreType.DMA,
        ],
    )
    def _sc_gather(w_hbm, idx_hbm, out_hbm, idx_v, rows_v, sem):
        wid = lax.axis_index("s") * _NC + lax.axis_index("c")
        base = wid * _BPW
        for c in range(_BPW // _CH):
            off = base + c * _CH
            pltpu.sync_copy(idx_hbm.at[pl.ds(off, _CH)], idx_v)
            pltpu.async_copy(w_hbm.at[idx_v], rows_v, sem).wait()
            pltpu.sync_copy(rows_v, out_hbm.at[pl.ds(off, _CH)])

    return _sc_gather


def kernel(z, W):
    idx, loss = _distance_argmin(z, W)
    zq_rows = _build_sc_gather()(W, idx)
    z_q_out = zq_rows.reshape(BATCH, SEQ, EMB_DIM).transpose(0, 2, 1)
    return (z_q_out, loss, (None, None, idx))


# speedup vs baseline: 1.5576x; 1.0385x over previous
import functools

import jax
import jax.numpy as jnp
from jax import lax
from jax.experimental import pallas as pl
from jax.experimental.pallas import tpu as pltpu
from jax.experimental.pallas import tpu_sc as plsc

N_EMB = 8192
EMB_DIM = 256
BATCH = 16
SEQ = 1024
M = BATCH * SEQ
BETA = 0.25

BL = 512
BK = 1024


_CHUNKS = ((0, 2736), (2736, 2736), (5472, 2720))


def _bf16_round(x):
    return x.astype(jnp.bfloat16).astype(jnp.float32)


def _dist_body(z_ref, w_ref, idx_ref, loss_ref, w2_ref, acc):
    first = (pl.program_id(0) == 0) & (pl.program_id(1) == 0)

    @pl.when(first)
    def _():
        for (s, c) in _CHUNKS:
            wb = w_ref[s:s + c]
            w2_ref[s:s + c] = jnp.sum(wb * wb, axis=1, keepdims=True)

    zb = z_ref[0]
    z2 = jnp.sum(zb * zb, axis=0, keepdims=True)
    rmin = ridx = rsel = None
    for (s, c) in _CHUNKS:
        wb = w_ref[s:s + c]
        mm = lax.dot_general(wb, zb, (((1,), (0,)), ((), ())),
                             preferred_element_type=jnp.float32)
        w2 = w2_ref[s:s + c]
        d = (z2 + w2) - 2.0 * mm
        lmin = jnp.min(d, axis=0, keepdims=True)
        iota = lax.broadcasted_iota(jnp.int32, (c, BL), 0)
        lidx = jnp.min(jnp.where(d == lmin, iota, N_EMB), axis=0,
                       keepdims=True) + s
        if rmin is None:
            ridx, rsel = lidx, lmin
            rmin = _bf16_round(lmin)
        else:
            keep = (rmin < lmin) | ((rmin == lmin) & (ridx < lidx))
            ridx = jnp.where(keep, ridx, lidx)
            rsel = jnp.where(keep, rsel, lmin)
            rmin = _bf16_round(jnp.where(keep, rmin, lmin))
    idx_ref[0] = ridx
    partial = jnp.sum(rsel)
    total = jnp.where(first, partial, acc[0] + partial)
    acc[0] = total
    loss_ref[0, 0] = total * ((1.0 + BETA) / (M * EMB_DIM))


def _distance_argmin(z, w):
    grid = (BATCH, SEQ // BL)
    idx, loss = pl.pallas_call(
        _dist_body,
        grid=grid,
        in_specs=[
            pl.BlockSpec((1, EMB_DIM, BL), lambda b, l: (b, 0, l)),
            pl.BlockSpec((N_EMB, EMB_DIM), lambda b, l: (0, 0)),
        ],
        out_specs=[
            pl.BlockSpec((1, 1, BL), lambda b, l: (b * (SEQ // BL) + l, 0, 0)),
            pl.BlockSpec((1, 1), lambda b, l: (0, 0),
                         memory_space=pltpu.SMEM),
        ],
        out_shape=[
            jax.ShapeDtypeStruct((M // BL, 1, BL), jnp.int32),
            jax.ShapeDtypeStruct((1, 1), jnp.float32),
        ],
        scratch_shapes=[
            pltpu.VMEM((N_EMB, 1), jnp.float32),
            pltpu.SMEM((1,), jnp.float32),
        ],
    )(z, w)
    return idx.reshape(M), loss[0, 0]


_NC = 2
_NS = 16
_NW = _NC * _NS
_BPW = M // _NW
_CH = 128

@functools.lru_cache(maxsize=None)
def _build_sc_gather():
    mesh = plsc.VectorSubcoreMesh(core_axis_name="c", subcore_axis_name="s")

    @functools.partial(
        pl.kernel,
        mesh=mesh,
        out_type=jax.ShapeDtypeStruct((M, EMB_DIM), jnp.float32),
        scratch_types=[
            pltpu.VMEM((_CH,), jnp.int32),
            pltpu.VMEM((_CH, EMB_DIM), jnp.float32),
            pltpu.SemaphoreType.DMA,
        ],
    )
    def _sc_gather(w_hbm, idx_hbm, out_hbm, idx_v, rows_v, sem):
        wid = lax.axis_index("s") * _NC + lax.axis_index("c")
        base = wid * _BPW
        for c in range(_BPW // _CH):
            off = base + c * _CH
            pltpu.sync_copy(idx_hbm.at[pl.ds(off, _CH)], idx_v)
            pltpu.async_copy(w_hbm.at[idx_v], rows_v, sem).wait()
            pltpu.sync_copy(rows_v, out_hbm.at[pl.ds(off, _CH)])

    return _sc_gather


def kernel(z, W):
    idx, loss = _distance_argmin(z, W)
    zq_rows = _build_sc_gather()(W, idx)
    z_q_out = zq_rows.reshape(BATCH, SEQ, EMB_DIM).transpose(0, 2, 1)
    return (z_q_out, loss, (None, None, idx))


# speedup vs baseline: 1.6002x; 1.0274x over previous
import functools

import jax
import jax.numpy as jnp
from jax import lax
from jax.experimental import pallas as pl
from jax.experimental.pallas import tpu as pltpu
from jax.experimental.pallas import tpu_sc as plsc

N_EMB = 8192
EMB_DIM = 256
BATCH = 16
SEQ = 1024
M = BATCH * SEQ
BETA = 0.25

BL = 512
BK = 1024


_CHUNKS = ((0, 2736), (2736, 2736), (5472, 2720))


def _bf16_round(x):
    return x.astype(jnp.bfloat16).astype(jnp.float32)


def _dist_body(z_ref, w_ref, idx_ref, loss_ref, w2_ref, wn_ref, fi_ref, acc):
    first = (pl.program_id(0) == 0) & (pl.program_id(1) == 0)

    @pl.when(first)
    def _():
        for (s, c) in _CHUNKS:
            wb = w_ref[s:s + c]
            w2_ref[s:s + c] = jnp.sum(wb * wb, axis=1, keepdims=True)
            wn_ref[s:s + c] = wb * -2.0
        fi_ref[...] = lax.broadcasted_iota(
            jnp.int32, (_CHUNKS[0][1], BL), 0).astype(jnp.float32)

    zb = z_ref[0]
    z2 = jnp.sum(zb * zb, axis=0, keepdims=True)
    rmin = ridx = rsel = None
    for (s, c) in _CHUNKS:
        mmn = lax.dot_general(wn_ref[s:s + c], zb, (((1,), (0,)), ((), ())),
                              preferred_element_type=jnp.float32)
        w2 = w2_ref[s:s + c]
        d = (z2 + w2) + mmn
        lmin = jnp.min(d, axis=0, keepdims=True)
        fiota = fi_ref[0:c]
        lidx = (jnp.min(jnp.where(d == lmin, fiota, float(N_EMB)), axis=0,
                        keepdims=True).astype(jnp.int32) + s)
        if rmin is None:
            ridx, rsel = lidx, lmin
            rmin = _bf16_round(lmin)
        else:
            keep = (rmin < lmin) | ((rmin == lmin) & (ridx < lidx))
            ridx = jnp.where(keep, ridx, lidx)
            rsel = jnp.where(keep, rsel, lmin)
            rmin = _bf16_round(jnp.where(keep, rmin, lmin))
    idx_ref[0] = ridx
    partial = jnp.sum(rsel)
    total = jnp.where(first, partial, acc[0] + partial)
    acc[0] = total
    loss_ref[0, 0] = total * ((1.0 + BETA) / (M * EMB_DIM))


def _distance_argmin(z, w):
    grid = (BATCH, SEQ // BL)
    idx, loss = pl.pallas_call(
        _dist_body,
        grid=grid,
        in_specs=[
            pl.BlockSpec((1, EMB_DIM, BL), lambda b, l: (b, 0, l)),
            pl.BlockSpec((N_EMB, EMB_DIM), lambda b, l: (0, 0)),
        ],
        out_specs=[
            pl.BlockSpec((1, 1, BL), lambda b, l: (b * (SEQ // BL) + l, 0, 0)),
            pl.BlockSpec((1, 1), lambda b, l: (0, 0),
                         memory_space=pltpu.SMEM),
        ],
        out_shape=[
            jax.ShapeDtypeStruct((M // BL, 1, BL), jnp.int32),
            jax.ShapeDtypeStruct((1, 1), jnp.float32),
        ],
        scratch_shapes=[
            pltpu.VMEM((N_EMB, 1), jnp.float32),
            pltpu.VMEM((N_EMB, EMB_DIM), jnp.float32),
            pltpu.VMEM((_CHUNKS[0][1], BL), jnp.float32),
            pltpu.SMEM((1,), jnp.float32),
        ],
    )(z, w)
    return idx.reshape(M), loss[0, 0]


_NC = 2
_NS = 16
_NW = _NC * _NS
_BPW = M // _NW
_CH = 128

@functools.lru_cache(maxsize=None)
def _build_sc_gather():
    mesh = plsc.VectorSubcoreMesh(core_axis_name="c", subcore_axis_name="s")

    @functools.partial(
        pl.kernel,
        mesh=mesh,
        out_type=jax.ShapeDtypeStruct((M, EMB_DIM), jnp.float32),
        scratch_types=[
            pltpu.VMEM((_CH,), jnp.int32),
            pltpu.VMEM((_CH, EMB_DIM), jnp.float32),
            pltpu.SemaphoreType.DMA,
        ],
    )
    def _sc_gather(w_hbm, idx_hbm, out_hbm, idx_v, rows_v, sem):
        wid = lax.axis_index("s") * _NC + lax.axis_index("c")
        base = wid * _BPW
        for c in range(_BPW // _CH):
            off = base + c * _CH
            pltpu.sync_copy(idx_hbm.at[pl.ds(off, _CH)], idx_v)
            pltpu.async_copy(w_hbm.at[idx_v], rows_v, sem).wait()
            pltpu.sync_copy(rows_v, out_hbm.at[pl.ds(off, _CH)])

    return _sc_gather


def kernel(z, W):
    idx, loss = _distance_argmin(z, W)
    zq_rows = _build_sc_gather()(W, idx)
    z_q_out = zq_rows.reshape(BATCH, SEQ, EMB_DIM).transpose(0, 2, 1)
    return (z_q_out, loss, (None, None, idx))


# speedup vs baseline: 2.3702x; 1.4811x over previous
import functools

import jax
import jax.numpy as jnp
from jax import lax
from jax.experimental import pallas as pl
from jax.experimental.pallas import tpu as pltpu
from jax.experimental.pallas import tpu_sc as plsc

N_EMB = 8192
EMB_DIM = 256
BATCH = 16
SEQ = 1024
M = BATCH * SEQ
BETA = 0.25

BL = 512
BK = 1024


_CHUNKS = ((0, 2736), (2736, 2736), (5472, 2720))


def _bf16_round(x):
    return x.astype(jnp.bfloat16).astype(jnp.float32)


_GR = 16


def _dist_body(z_ref, w_ref, idx_ref, loss_ref, w2_ref, wn_ref,
               mm0_ref, mm1_ref, acc):
    first = (pl.program_id(0) == 0) & (pl.program_id(1) == 0)

    @pl.when(first)
    def _():
        for (s, c) in _CHUNKS:
            wb = w_ref[s:s + c]
            w2_ref[s:s + c] = jnp.sum(wb * wb, axis=1, keepdims=True)
            wn_ref[s:s + c] = wb * -2.0

    zb = z_ref[0]
    z2 = jnp.sum(zb * zb, axis=0, keepdims=True)
    mm_refs = (mm0_ref, mm1_ref)
    rmin = ridx = rsel = None
    for ci, (s, c) in enumerate(_CHUNKS):
        mmr = mm_refs[ci % 2]
        mmr[0:c] = lax.dot_general(wn_ref[s:s + c], zb, (((1,), (0,)), ((), ())),
                                   preferred_element_type=jnp.float32)
        accv = jnp.full((_GR, BL), jnp.inf, jnp.float32)
        accg = jnp.zeros((_GR, BL), jnp.int32)
        for g in range(c // _GR):
            mmg = mmr[g * _GR:(g + 1) * _GR]
            w2g = w2_ref[s + g * _GR:s + (g + 1) * _GR]
            d = (z2 + w2g) + mmg
            lt = d < accv
            accv = jnp.where(lt, d, accv)
            accg = jnp.where(lt, g, accg)
        r_iota = lax.broadcasted_iota(jnp.int32, (_GR, BL), 0)
        gidx = accg * _GR + r_iota + s
        lmin = jnp.min(accv, axis=0, keepdims=True)
        lidx = jnp.min(jnp.where(accv == lmin, gidx, N_EMB), axis=0,
                       keepdims=True)
        if rmin is None:
            ridx, rsel = lidx, lmin
            rmin = _bf16_round(lmin)
        else:
            keep = (rmin < lmin) | ((rmin == lmin) & (ridx < lidx))
            ridx = jnp.where(keep, ridx, lidx)
            rsel = jnp.where(keep, rsel, lmin)
            rmin = _bf16_round(jnp.where(keep, rmin, lmin))
    idx_ref[0] = ridx
    partial = jnp.sum(rsel)
    total = jnp.where(first, partial, acc[0] + partial)
    acc[0] = total
    loss_ref[0, 0] = total * ((1.0 + BETA) / (M * EMB_DIM))


def _distance_argmin(z, w):
    grid = (BATCH, SEQ // BL)
    idx, loss = pl.pallas_call(
        _dist_body,
        grid=grid,
        in_specs=[
            pl.BlockSpec((1, EMB_DIM, BL), lambda b, l: (b, 0, l)),
            pl.BlockSpec((N_EMB, EMB_DIM), lambda b, l: (0, 0)),
        ],
        out_specs=[
            pl.BlockSpec((1, 1, BL), lambda b, l: (b * (SEQ // BL) + l, 0, 0)),
            pl.BlockSpec((1, 1), lambda b, l: (0, 0),
                         memory_space=pltpu.SMEM),
        ],
        out_shape=[
            jax.ShapeDtypeStruct((M // BL, 1, BL), jnp.int32),
            jax.ShapeDtypeStruct((1, 1), jnp.float32),
        ],
        scratch_shapes=[
            pltpu.VMEM((N_EMB, 1), jnp.float32),
            pltpu.VMEM((N_EMB, EMB_DIM), jnp.float32),
            pltpu.VMEM((_CHUNKS[0][1], BL), jnp.float32),
            pltpu.VMEM((_CHUNKS[0][1], BL), jnp.float32),
            pltpu.SMEM((1,), jnp.float32),
        ],
    )(z, w)
    return idx.reshape(M), loss[0, 0]


_NC = 2
_NS = 16
_NW = _NC * _NS
_BPW = M // _NW
_CH = 128

@functools.lru_cache(maxsize=None)
def _build_sc_gather():
    mesh = plsc.VectorSubcoreMesh(core_axis_name="c", subcore_axis_name="s")

    @functools.partial(
        pl.kernel,
        mesh=mesh,
        out_type=jax.ShapeDtypeStruct((M, EMB_DIM), jnp.float32),
        scratch_types=[
            pltpu.VMEM((_CH,), jnp.int32),
            pltpu.VMEM((_CH, EMB_DIM), jnp.float32),
            pltpu.SemaphoreType.DMA,
        ],
    )
    def _sc_gather(w_hbm, idx_hbm, out_hbm, idx_v, rows_v, sem):
        wid = lax.axis_index("s") * _NC + lax.axis_index("c")
        base = wid * _BPW
        for c in range(_BPW // _CH):
            off = base + c * _CH
            pltpu.sync_copy(idx_hbm.at[pl.ds(off, _CH)], idx_v)
            pltpu.async_copy(w_hbm.at[idx_v], rows_v, sem).wait()
            pltpu.sync_copy(rows_v, out_hbm.at[pl.ds(off, _CH)])

    return _sc_gather


def kernel(z, W):
    idx, loss = _distance_argmin(z, W)
    zq_rows = _build_sc_gather()(W, idx)
    z_q_out = zq_rows.reshape(BATCH, SEQ, EMB_DIM).transpose(0, 2, 1)
    return (z_q_out, loss, (None, None, idx))


# speedup vs baseline: 2.4580x; 1.0371x over previous
import functools

import jax
import jax.numpy as jnp
from jax import lax
from jax.experimental import pallas as pl
from jax.experimental.pallas import tpu as pltpu
from jax.experimental.pallas import tpu_sc as plsc

N_EMB = 8192
EMB_DIM = 256
BATCH = 16
SEQ = 1024
M = BATCH * SEQ
BETA = 0.25

BL = 1024
BK = 1024


_CHUNKS = ((0, 2736), (2736, 2736), (5472, 2720))


def _bf16_round(x):
    return x.astype(jnp.bfloat16).astype(jnp.float32)


_GR = 16


def _dist_body(z_ref, w_ref, idx_ref, loss_ref, w2_ref, wn_ref,
               mm0_ref, mm1_ref, acc):
    first = (pl.program_id(0) == 0) & (pl.program_id(1) == 0)

    @pl.when(first)
    def _():
        for (s, c) in _CHUNKS:
            wb = w_ref[s:s + c]
            w2_ref[s:s + c] = jnp.sum(wb * wb, axis=1, keepdims=True)
            wn_ref[s:s + c] = wb * -2.0

    zb = z_ref[0]
    z2 = jnp.sum(zb * zb, axis=0, keepdims=True)
    mm_refs = (mm0_ref, mm1_ref)
    rmin = ridx = rsel = None
    for ci, (s, c) in enumerate(_CHUNKS):
        mmr = mm_refs[ci % 2]
        mmr[0:c] = lax.dot_general(wn_ref[s:s + c], zb, (((1,), (0,)), ((), ())),
                                   preferred_element_type=jnp.float32)
        accv = jnp.full((_GR, BL), jnp.inf, jnp.float32)
        accg = jnp.zeros((_GR, BL), jnp.int32)
        for g in range(c // _GR):
            mmg = mmr[g * _GR:(g + 1) * _GR]
            w2g = w2_ref[s + g * _GR:s + (g + 1) * _GR]
            d = (z2 + w2g) + mmg
            lt = d < accv
            accv = jnp.where(lt, d, accv)
            accg = jnp.where(lt, g, accg)
        r_iota = lax.broadcasted_iota(jnp.int32, (_GR, BL), 0)
        gidx = accg * _GR + r_iota + s
        lmin = jnp.min(accv, axis=0, keepdims=True)
        lidx = jnp.min(jnp.where(accv == lmin, gidx, N_EMB), axis=0,
                       keepdims=True)
        if rmin is None:
            ridx, rsel = lidx, lmin
            rmin = _bf16_round(lmin)
        else:
            keep = (rmin < lmin) | ((rmin == lmin) & (ridx < lidx))
            ridx = jnp.where(keep, ridx, lidx)
            rsel = jnp.where(keep, rsel, lmin)
            rmin = _bf16_round(jnp.where(keep, rmin, lmin))
    idx_ref[0] = ridx
    partial = jnp.sum(rsel)
    total = jnp.where(first, partial, acc[0] + partial)
    acc[0] = total
    loss_ref[0, 0] = total * ((1.0 + BETA) / (M * EMB_DIM))


def _distance_argmin(z, w):
    grid = (BATCH, SEQ // BL)
    idx, loss = pl.pallas_call(
        _dist_body,
        grid=grid,
        in_specs=[
            pl.BlockSpec((1, EMB_DIM, BL), lambda b, l: (b, 0, l)),
            pl.BlockSpec((N_EMB, EMB_DIM), lambda b, l: (0, 0)),
        ],
        out_specs=[
            pl.BlockSpec((1, 1, BL), lambda b, l: (b * (SEQ // BL) + l, 0, 0)),
            pl.BlockSpec((1, 1), lambda b, l: (0, 0),
                         memory_space=pltpu.SMEM),
        ],
        out_shape=[
            jax.ShapeDtypeStruct((M // BL, 1, BL), jnp.int32),
            jax.ShapeDtypeStruct((1, 1), jnp.float32),
        ],
        scratch_shapes=[
            pltpu.VMEM((N_EMB, 1), jnp.float32),
            pltpu.VMEM((N_EMB, EMB_DIM), jnp.float32),
            pltpu.VMEM((_CHUNKS[0][1], BL), jnp.float32),
            pltpu.VMEM((_CHUNKS[0][1], BL), jnp.float32),
            pltpu.SMEM((1,), jnp.float32),
        ],
    )(z, w)
    return idx.reshape(M), loss[0, 0]


_NC = 2
_NS = 16
_NW = _NC * _NS
_BPW = M // _NW
_CH = 128

@functools.lru_cache(maxsize=None)
def _build_sc_gather():
    mesh = plsc.VectorSubcoreMesh(core_axis_name="c", subcore_axis_name="s")

    @functools.partial(
        pl.kernel,
        mesh=mesh,
        out_type=jax.ShapeDtypeStruct((M, EMB_DIM), jnp.float32),
        scratch_types=[
            pltpu.VMEM((_CH,), jnp.int32),
            pltpu.VMEM((_CH, EMB_DIM), jnp.float32),
            pltpu.SemaphoreType.DMA,
        ],
    )
    def _sc_gather(w_hbm, idx_hbm, out_hbm, idx_v, rows_v, sem):
        wid = lax.axis_index("s") * _NC + lax.axis_index("c")
        base = wid * _BPW
        for c in range(_BPW // _CH):
            off = base + c * _CH
            pltpu.sync_copy(idx_hbm.at[pl.ds(off, _CH)], idx_v)
            pltpu.async_copy(w_hbm.at[idx_v], rows_v, sem).wait()
            pltpu.sync_copy(rows_v, out_hbm.at[pl.ds(off, _CH)])

    return _sc_gather


def kernel(z, W):
    idx, loss = _distance_argmin(z, W)
    zq_rows = _build_sc_gather()(W, idx)
    z_q_out = zq_rows.reshape(BATCH, SEQ, EMB_DIM).transpose(0, 2, 1)
    return (z_q_out, loss, (None, None, idx))


# speedup vs baseline: 2.4950x; 1.0150x over previous
import functools

import jax
import jax.numpy as jnp
from jax import lax
from jax.experimental import pallas as pl
from jax.experimental.pallas import tpu as pltpu
from jax.experimental.pallas import tpu_sc as plsc

N_EMB = 8192
EMB_DIM = 256
BATCH = 16
SEQ = 1024
M = BATCH * SEQ
BETA = 0.25

BL = 1024
BK = 1024


_CHUNKS = ((0, 2736), (2736, 2736), (5472, 2720))


def _bf16_round(x):
    return x.astype(jnp.bfloat16).astype(jnp.float32)


_GR = 16


def _dist_body(z_ref, w_ref, idx_ref, loss_ref, w2_ref, wn_ref,
               mm0_ref, mm1_ref, acc):
    first = (pl.program_id(0) == 0) & (pl.program_id(1) == 0)

    @pl.when(first)
    def _():
        for (s, c) in _CHUNKS:
            wb = w_ref[s:s + c]
            w2_ref[s:s + c] = jnp.sum(wb * wb, axis=1, keepdims=True)
            wn_ref[s:s + c] = wb * -2.0

    zb = z_ref[0]
    z2 = jnp.sum(zb * zb, axis=0, keepdims=True)
    mm_refs = (mm0_ref, mm1_ref)
    rmin = ridx = rsel = None
    for ci, (s, c) in enumerate(_CHUNKS):
        mmr = mm_refs[ci % 2]
        mmr[0:c] = lax.dot_general(wn_ref[s:s + c], zb, (((1,), (0,)), ((), ())),
                                   preferred_element_type=jnp.float32)
        accv = jnp.full((_GR, BL), jnp.inf, jnp.float32)
        accg = jnp.zeros((_GR, BL), jnp.int32)
        for g in range(c // _GR):
            mmg = mmr[g * _GR:(g + 1) * _GR]
            w2g = w2_ref[s + g * _GR:s + (g + 1) * _GR]
            d = (z2 + w2g) + mmg
            lt = d < accv
            accv = jnp.where(lt, d, accv)
            accg = jnp.where(lt, g, accg)
        r_iota = lax.broadcasted_iota(jnp.int32, (_GR, BL), 0)
        gidx = accg * _GR + r_iota + s
        lmin = jnp.min(accv, axis=0, keepdims=True)
        lidx = jnp.min(jnp.where(accv == lmin, gidx, N_EMB), axis=0,
                       keepdims=True)
        if rmin is None:
            ridx, rsel = lidx, lmin
            rmin = _bf16_round(lmin)
        else:
            keep = (rmin < lmin) | ((rmin == lmin) & (ridx < lidx))
            ridx = jnp.where(keep, ridx, lidx)
            rsel = jnp.where(keep, rsel, lmin)
            rmin = _bf16_round(jnp.where(keep, rmin, lmin))
    idx_ref[0] = ridx
    partial = jnp.sum(rsel)
    total = jnp.where(first, partial, acc[0] + partial)
    acc[0] = total
    loss_ref[0, 0] = total * ((1.0 + BETA) / (M * EMB_DIM))


def _distance_argmin(z, w):
    grid = (BATCH, SEQ // BL)
    idx, loss = pl.pallas_call(
        _dist_body,
        grid=grid,
        in_specs=[
            pl.BlockSpec((1, EMB_DIM, BL), lambda b, l: (b, 0, l)),
            pl.BlockSpec((N_EMB, EMB_DIM), lambda b, l: (0, 0)),
        ],
        out_specs=[
            pl.BlockSpec((1, 1, BL), lambda b, l: (b * (SEQ // BL) + l, 0, 0)),
            pl.BlockSpec((1, 1), lambda b, l: (0, 0),
                         memory_space=pltpu.SMEM),
        ],
        out_shape=[
            jax.ShapeDtypeStruct((M // BL, 1, BL), jnp.int32),
            jax.ShapeDtypeStruct((1, 1), jnp.float32),
        ],
        scratch_shapes=[
            pltpu.VMEM((N_EMB, 1), jnp.float32),
            pltpu.VMEM((N_EMB, EMB_DIM), jnp.float32),
            pltpu.VMEM((_CHUNKS[0][1], BL), jnp.float32),
            pltpu.VMEM((_CHUNKS[0][1], BL), jnp.float32),
            pltpu.SMEM((1,), jnp.float32),
        ],
    )(z, w)
    return idx.reshape(M), loss[0, 0]


_NC = 2
_NS = 16
_NW = _NC * _NS
_BPW = M // _NW
_CH = 128

@functools.lru_cache(maxsize=None)
def _build_sc_gather():
    mesh = plsc.VectorSubcoreMesh(core_axis_name="c", subcore_axis_name="s")

    @functools.partial(
        pl.kernel,
        mesh=mesh,
        out_type=jax.ShapeDtypeStruct((M, EMB_DIM), jnp.float32),
        scratch_types=[
            pltpu.VMEM((_CH,), jnp.int32),
            pltpu.VMEM((_CH,), jnp.int32),
            pltpu.VMEM((_CH, EMB_DIM), jnp.float32),
            pltpu.VMEM((_CH, EMB_DIM), jnp.float32),
            pltpu.SemaphoreType.DMA,
            pltpu.SemaphoreType.DMA,
            pltpu.SemaphoreType.DMA,
            pltpu.SemaphoreType.DMA,
        ],
    )
    def _sc_gather(w_hbm, idx_hbm, out_hbm, idx_v0, idx_v1, rows_v0, rows_v1,
                   gs0, gs1, os0, os1):
        wid = lax.axis_index("s") * _NC + lax.axis_index("c")
        base = wid * _BPW
        idx_bufs, row_bufs = (idx_v0, idx_v1), (rows_v0, rows_v1)
        gsems, osems = (gs0, gs1), (os0, os1)
        ncg = _BPW // _CH
        pltpu.sync_copy(idx_hbm.at[pl.ds(base, _CH)], idx_bufs[0])
        gath = [pltpu.async_copy(w_hbm.at[idx_bufs[0]], row_bufs[0], gsems[0])]
        outs = []
        for c in range(ncg):
            if c + 1 < ncg:
                if c >= 1:
                    outs[c - 1].wait()
                b = (c + 1) % 2
                pltpu.sync_copy(idx_hbm.at[pl.ds(base + (c + 1) * _CH, _CH)],
                                idx_bufs[b])
                gath.append(pltpu.async_copy(w_hbm.at[idx_bufs[b]],
                                             row_bufs[b], gsems[b]))
            gath[c].wait()
            outs.append(pltpu.async_copy(row_bufs[c % 2],
                                         out_hbm.at[pl.ds(base + c * _CH, _CH)],
                                         osems[c % 2]))
        outs[-2].wait()
        outs[-1].wait()

    return _sc_gather


def kernel(z, W):
    idx, loss = _distance_argmin(z, W)
    zq_rows = _build_sc_gather()(W, idx)
    z_q_out = zq_rows.reshape(BATCH, SEQ, EMB_DIM).transpose(0, 2, 1)
    return (z_q_out, loss, (None, None, idx))
